# baseline (speedup 1.0000x reference)
import jax
import jax.numpy as jnp
from jax import lax
from jax.experimental import pallas as pl
from jax.experimental.pallas import tpu as pltpu
from jax.experimental.pallas import tpu_sc as plsc

N = 10000
E = 320000
DIN = 128
H = 64
NG = 256

NC = 2
NS = 16
NW = NC * NS
LANES = 16

K = 128
CH_TOTAL = E // K
CH_W = CH_TOTAL // NW
EXTRA = CH_TOTAL - CH_W * NW
EDGES_W = CH_W * K

NPAD = 10240
DEG_T = NPAD // NS
ROWS_T = NPAD // NS
NB = 384
BT = NB // NS
GT = NG // NS

NP2 = NPAD // 2
XP = N // 2
HP = 2 * H
NGP = NG // 2


def _mesh():
    return plsc.VectorSubcoreMesh(core_axis_name="c", subcore_axis_name="s")


_SC_PARAMS = pltpu.CompilerParams(use_tc_tiling_on_sc=False)


def _fill_ones(ones_v):
    for k in range(K // LANES):
        ones_v[pl.ds(k * LANES, LANES)] = jnp.ones((LANES,), jnp.float32)


def _stage_chunk(src_ref, off, dst_ref):
    for k in range(K // LANES):
        dst_ref[pl.ds(k * LANES, LANES)] = src_ref[pl.ds(off + k * LANES, LANES)]


def _sc_deg(ei_hbm, z640, degw, d_all, d0, ones_v, dv_v, dexp, deg_sp):
    c = lax.axis_index("c")
    s = lax.axis_index("s")
    wid = c * NS + s
    pltpu.sync_copy(ei_hbm.at[pl.ds(E + wid * EDGES_W, EDGES_W)], d_all)
    _fill_ones(ones_v)
    pltpu.sync_copy(z640, deg_sp.at[pl.ds(s * DEG_T, DEG_T)])
    plsc.subcore_barrier()

    def body(j, carry):
        _stage_chunk(d_all, j * K, d0)
        pltpu.sync_copy(ones_v, deg_sp.at[d0], add=True)
        return carry

    lax.fori_loop(0, CH_W, body, 0)

    @pl.when(wid < EXTRA)
    def _():
        pltpu.sync_copy(ei_hbm.at[pl.ds(E + (CH_W * NW + wid) * K, K)], d0)
        pltpu.sync_copy(ones_v, deg_sp.at[d0], add=True)

    plsc.subcore_barrier()
    pltpu.sync_copy(deg_sp.at[pl.ds(s * DEG_T, DEG_T)], dv_v)

    def ebody(g, carry):
        vec = dv_v[pl.ds(g * LANES, LANES)]
        for l in range(LANES):
            bv = jnp.full((LANES,), vec[l], jnp.float32)
            for k in range(H // LANES):
                dexp[g * LANES + l, pl.ds(k * LANES, LANES)] = bv
        return carry

    lax.fori_loop(0, DEG_T // LANES, ebody, 0)
    pltpu.sync_copy(dexp, degw.at[pl.ds(c * NPAD + s * DEG_T, DEG_T), :])


NBUF = 6
LG = 3


def _sc_edge(gs, ei_hbm, accp,
             s_all, d_all, rows, sbufs, dbufs, acc_sp, gsems, ssems):
    c = lax.axis_index("c")
    s = lax.axis_index("s")
    wid = c * NS + s
    base_e = wid * EDGES_W
    pltpu.sync_copy(ei_hbm.at[pl.ds(base_e, EDGES_W)], s_all)
    pltpu.sync_copy(ei_hbm.at[pl.ds(E + base_e, EDGES_W)], d_all)
    pltpu.sync_copy(gs.at[pl.ds(s * ROWS_T, ROWS_T), :],
                    acc_sp.at[pl.ds(s * ROWS_T, ROWS_T), :])
    plsc.subcore_barrier()

    def issue_gather(j, b):
        _stage_chunk(s_all, j * K, sbufs[b])
        _stage_chunk(d_all, j * K, dbufs[b])
        pltpu.async_copy(gs.at[sbufs[b]], rows[b], gsems[b])

    def wait_gather(b):
        pltpu.make_async_copy(gs.at[pl.ds(0, K), :], rows[b], gsems[b]).wait()

    def issue_scatter(b):
        pltpu.async_copy(rows[b], acc_sp.at[dbufs[b]], ssems[b], add=True)

    def wait_scatter(b):
        pltpu.make_async_copy(rows[b], acc_sp.at[dbufs[b]], ssems[b]).wait()

    for t in range(LG):
        issue_gather(t, t)

    def body(jo, carry):
        for b in range(NBUF):
            j = NBUF * jo + b
            jg = j + LG
            bg = (b + LG) % NBUF
            if b < LG:
                @pl.when((jo > 0) & (jg < CH_W))
                def _():
                    wait_scatter(bg)
            else:
                @pl.when(jg < CH_W)
                def _():
                    wait_scatter(bg)

            @pl.when(jg < CH_W)
            def _():
                issue_gather(jg, bg)

            wait_gather(b)
            issue_scatter(b)
        return carry

    lax.fori_loop(0, CH_W // NBUF, body, 0)
    for b in range(NBUF):
        wait_scatter(b)

    @pl.when(wid < EXTRA)
    def _():
        pltpu.sync_copy(ei_hbm.at[pl.ds((CH_W * NW + wid) * K, K)], sbufs[0])
        pltpu.async_copy(gs.at[sbufs[0]], rows[0], gsems[0]).wait()
        pltpu.sync_copy(ei_hbm.at[pl.ds(E + (CH_W * NW + wid) * K, K)], dbufs[0])
        pltpu.sync_copy(rows[0], acc_sp.at[dbufs[0]], add=True)

    plsc.subcore_barrier()
    pltpu.sync_copy(acc_sp.at[pl.ds(s * ROWS_T, ROWS_T), :],
                    accp.at[pl.ds(c * NPAD + s * ROWS_T, ROWS_T), :])


def _sc_pool(h4p, bip, zp, z24, poolp, cntw,
             b0, rows, ones_v, cv, cexp, pool_sp, cnt_sp):
    c = lax.axis_index("c")
    s = lax.axis_index("s")
    wid = c * NS + s
    pltpu.sync_copy(zp, pool_sp.at[pl.ds(s * BT, BT), :])
    pltpu.sync_copy(z24, cnt_sp.at[pl.ds(s * BT, BT)])
    _fill_ones(ones_v)
    plsc.subcore_barrier()

    def do_chunk(ch):
        base = ch * K
        pltpu.sync_copy(bip.at[pl.ds(base, K)], b0)
        pltpu.sync_copy(h4p.at[pl.ds(base, K), :], rows)
        pltpu.sync_copy(rows, pool_sp.at[b0], add=True)
        pltpu.sync_copy(ones_v, cnt_sp.at[b0], add=True)

    do_chunk(wid)
    do_chunk(wid + NW)

    @pl.when(wid < (NPAD // K) - 2 * NW)
    def _():
        do_chunk(wid + 2 * NW)

    plsc.subcore_barrier()
    pltpu.sync_copy(pool_sp.at[pl.ds(s * GT, GT), :],
                    poolp.at[pl.ds(c * NG + s * GT, GT), :])
    pltpu.sync_copy(cnt_sp.at[pl.ds(s * GT, GT)], cv)
    cvec = cv[pl.ds(0, LANES)]
    for i in range(GT):
        bv = jnp.full((LANES,), cvec[i], jnp.float32)
        for k in range(H // LANES):
            cexp[i, pl.ds(k * LANES, LANES)] = bv
    pltpu.sync_copy(cexp, cntw.at[pl.ds(c * NG + s * GT, GT), :])


def _tc_prep_mm0(d0w_ref, d1w_ref, xp_ref, wbd_ref, gs_ref, dvw_ref):
    dvw = lax.rsqrt(d0w_ref[...] + d1w_ref[...] + 1.0)
    dvw_ref[...] = dvw
    gs_ref[pl.ds(0, XP), :] = dvw[:XP] * jnp.dot(
        xp_ref[...], wbd_ref[...], preferred_element_type=jnp.float32)
    gs_ref[pl.ds(XP, NP2 - XP), :] = jnp.zeros((NP2 - XP, HP), jnp.float32)


def _tc_layer(accp_ref, gsp_ref, dvw_ref, b_ref, wbd_ref, out_ref):
    a = accp_ref[0:NP2, :] + accp_ref[NP2:2 * NP2, :] - gsp_ref[...]
    h = jnp.maximum(dvw_ref[...] * a + b_ref[...], 0.0)
    out_ref[...] = dvw_ref[...] * jnp.dot(
        h, wbd_ref[...], preferred_element_type=jnp.float32)


def _tc_final(accp_ref, gsp_ref, dvw_ref, b_ref, out_ref):
    a = accp_ref[0:NP2, :] + accp_ref[NP2:2 * NP2, :] - gsp_ref[...]
    h = jnp.maximum(dvw_ref[...] * a + b_ref[...], 0.0)
    out_ref[...] = h
    out_ref[pl.ds(XP, NP2 - XP), :] = jnp.zeros((NP2 - XP, HP), jnp.float32)


def _tc_head(pp_ref, cw_ref, wbd_ref, b_ref, out_ref, hid_ref):
    sums = pp_ref[0:NGP, :] + pp_ref[NGP:2 * NGP, :]
    cnt = cw_ref[0:NGP, :] + cw_ref[NGP:2 * NGP, :]
    hid = sums / jnp.maximum(cnt, 1.0)
    hid_ref[...] = hid
    out_ref[...] = jnp.dot(
        hid, wbd_ref[...], preferred_element_type=jnp.float32) + b_ref[...]


def kernel(x, edge_index, batch_index, W0, b0, W1, b1, W2, b2, W3, b3,
           W_out, b_out):
    f32 = jnp.float32
    ei_flat = edge_index.reshape(2 * E)
    bip = jnp.concatenate(
        [batch_index, jnp.full((NPAD - N,), NG, jnp.int32)])
    z640 = jnp.zeros((DEG_T,), f32)
    zp = jnp.zeros((BT, H), f32)
    z24 = jnp.zeros((BT,), f32)

    def bd(W):
        fi, fo = W.shape
        z = jnp.zeros((fi, fo), f32)
        return jnp.concatenate(
            [jnp.concatenate([W, z], axis=1),
             jnp.concatenate([z, W], axis=1)], axis=0)

    deg_call = pl.kernel(
        _sc_deg,
        out_type=jax.ShapeDtypeStruct((NC * NPAD, H), f32),
        mesh=_mesh(),
        compiler_params=_SC_PARAMS,
        scratch_types=[
            pltpu.VMEM((EDGES_W,), jnp.int32),
            pltpu.VMEM((K,), jnp.int32),
            pltpu.VMEM((K,), f32),
            pltpu.VMEM((DEG_T,), f32),
            pltpu.VMEM((DEG_T, H), f32),
            pltpu.VMEM_SHARED((NPAD,), f32),
        ],
    )
    degw = deg_call(ei_flat, z640)

    gs, dvw = pl.pallas_call(
        _tc_prep_mm0,
        out_shape=(
            jax.ShapeDtypeStruct((NP2, HP), f32),
            jax.ShapeDtypeStruct((NP2, HP), f32),
        ),
    )(degw[:NPAD].reshape(NP2, HP), degw[NPAD:].reshape(NP2, HP),
      x.reshape(XP, 2 * DIN), bd(W0))

    edge_call = pl.kernel(
        _sc_edge,
        out_type=jax.ShapeDtypeStruct((NC * NPAD, H), f32),
        mesh=_mesh(),
        compiler_params=_SC_PARAMS,
        scratch_types=[
            pltpu.VMEM((EDGES_W,), jnp.int32),
            pltpu.VMEM((EDGES_W,), jnp.int32),
            [pltpu.VMEM((K, H), f32) for _ in range(NBUF)],
            [pltpu.VMEM((K,), jnp.int32) for _ in range(NBUF)],
            [pltpu.VMEM((K,), jnp.int32) for _ in range(NBUF)],
            pltpu.VMEM_SHARED((NPAD, H), f32),
            [pltpu.SemaphoreType.DMA for _ in range(NBUF)],
            [pltpu.SemaphoreType.DMA for _ in range(NBUF)],
        ],
    )

    layer_call = pl.pallas_call(
        _tc_layer,
        out_shape=jax.ShapeDtypeStruct((NP2, HP), f32),
    )

    for W_l, b_l in ((W1, b0), (W2, b1), (W3, b2)):
        accp = edge_call(gs.reshape(NPAD, H), ei_flat)
        gs = layer_call(accp.reshape(2 * NP2, HP), gs,
                        dvw, jnp.tile(b_l, 2).reshape(1, HP), bd(W_l))

    accp = edge_call(gs.reshape(NPAD, H), ei_flat)
    h4p = pl.pallas_call(
        _tc_final,
        out_shape=jax.ShapeDtypeStruct((NP2, HP), f32),
    )(accp.reshape(2 * NP2, HP), gs, dvw, jnp.tile(b3, 2).reshape(1, HP))

    pool_call = pl.kernel(
        _sc_pool,
        out_type=(
            jax.ShapeDtypeStruct((NC * NG, H), f32),
            jax.ShapeDtypeStruct((NC * NG, H), f32),
        ),
        mesh=_mesh(),
        compiler_params=_SC_PARAMS,
        scratch_types=[
            pltpu.VMEM((K,), jnp.int32),
            pltpu.VMEM((K, H), f32),
            pltpu.VMEM((K,), f32),
            pltpu.VMEM((GT,), f32),
            pltpu.VMEM((GT, H), f32),
            pltpu.VMEM_SHARED((NB, H), f32),
            pltpu.VMEM_SHARED((NB,), f32),
        ],
    )
    poolp, cntw = pool_call(h4p.reshape(NPAD, H), bip, zp, z24)

    out_p, hid_p = pl.pallas_call(
        _tc_head,
        out_shape=(
            jax.ShapeDtypeStruct((NGP, 2), f32),
            jax.ShapeDtypeStruct((NGP, HP), f32),
        ),
    )(poolp.reshape(NG, HP), cntw.reshape(NG, HP), bd(W_out),
      jnp.tile(b_out, 2).reshape(1, 2))
    return (out_p.reshape(NG, 1), hid_p.reshape(NG, H))


# speedup vs baseline: 1.0727x; 1.0727x over previous
import jax
import jax.numpy as jnp
from jax import lax
from jax.experimental import pallas as pl
from jax.experimental.pallas import tpu as pltpu
from jax.experimental.pallas import tpu_sc as plsc

N = 10000
E = 320000
DIN = 128
H = 64
NG = 256

NC = 2
NS = 16
NW = NC * NS
LANES = 16

K = 128
CH_TOTAL = E // K
CH_W = CH_TOTAL // NW
EXTRA = CH_TOTAL - CH_W * NW
EDGES_W = CH_W * K

NPAD = 10240
DEG_T = NPAD // NS
ROWS_T = NPAD // NS
NB = 384
BT = NB // NS
GT = NG // NS

NP2 = NPAD // 2
XP = N // 2
HP = 2 * H
NGP = NG // 2


def _mesh():
    return plsc.VectorSubcoreMesh(core_axis_name="c", subcore_axis_name="s")


_SC_PARAMS = pltpu.CompilerParams(use_tc_tiling_on_sc=False)


def _fill_ones(ones_v):
    for k in range(K // LANES):
        ones_v[pl.ds(k * LANES, LANES)] = jnp.ones((LANES,), jnp.float32)


def _stage_chunk(src_ref, off, dst_ref):
    for k in range(K // LANES):
        dst_ref[pl.ds(k * LANES, LANES)] = src_ref[pl.ds(off + k * LANES, LANES)]


def _sc_deg(ei_hbm, z640, degw, d_all, d0, ones_v, dv_v, dexp, deg_sp):
    c = lax.axis_index("c")
    s = lax.axis_index("s")
    wid = c * NS + s
    pltpu.sync_copy(ei_hbm.at[pl.ds(E + wid * EDGES_W, EDGES_W)], d_all)
    _fill_ones(ones_v)
    pltpu.sync_copy(z640, deg_sp.at[pl.ds(s * DEG_T, DEG_T)])
    plsc.subcore_barrier()

    def body(j, carry):
        _stage_chunk(d_all, j * K, d0)
        pltpu.sync_copy(ones_v, deg_sp.at[d0], add=True)
        return carry

    lax.fori_loop(0, CH_W, body, 0)

    @pl.when(wid < EXTRA)
    def _():
        pltpu.sync_copy(ei_hbm.at[pl.ds(E + (CH_W * NW + wid) * K, K)], d0)
        pltpu.sync_copy(ones_v, deg_sp.at[d0], add=True)

    plsc.subcore_barrier()
    pltpu.sync_copy(deg_sp.at[pl.ds(s * DEG_T, DEG_T)], dv_v)

    def ebody(g, carry):
        vec = dv_v[pl.ds(g * LANES, LANES)]
        for l in range(LANES):
            bv = jnp.full((LANES,), vec[l], jnp.float32)
            for k in range(H // LANES):
                dexp[g * LANES + l, pl.ds(k * LANES, LANES)] = bv
        return carry

    lax.fori_loop(0, DEG_T // LANES, ebody, 0)
    pltpu.sync_copy(dexp, degw.at[pl.ds(c * NPAD + s * DEG_T, DEG_T), :])


NBUF = 6
LG = 3


def _sc_edge(gs, ei_hbm, accp,
             s_all, d_all, rows, sbufs, dbufs, acc_sp, gsems, ssems):
    c = lax.axis_index("c")
    s = lax.axis_index("s")
    wid = c * NS + s
    base_e = wid * EDGES_W
    pltpu.sync_copy(ei_hbm.at[pl.ds(base_e, EDGES_W)], s_all)
    pltpu.sync_copy(ei_hbm.at[pl.ds(E + base_e, EDGES_W)], d_all)
    pltpu.sync_copy(gs.at[pl.ds(s * ROWS_T, ROWS_T), :],
                    acc_sp.at[pl.ds(s * ROWS_T, ROWS_T), :])
    plsc.subcore_barrier()

    def issue_gather(j, b):
        _stage_chunk(s_all, j * K, sbufs[b])
        _stage_chunk(d_all, j * K, dbufs[b])
        pltpu.async_copy(gs.at[sbufs[b]], rows[b], gsems[b])

    def wait_gather(b):
        pltpu.make_async_copy(gs.at[pl.ds(0, K), :], rows[b], gsems[b]).wait()

    def issue_scatter(b):
        pltpu.async_copy(rows[b], acc_sp.at[dbufs[b]], ssems[b], add=True)

    def wait_scatter(b):
        pltpu.make_async_copy(rows[b], acc_sp.at[dbufs[b]], ssems[b]).wait()

    for t in range(LG):
        issue_gather(t, t)

    def body(jo, carry):
        for b in range(NBUF):
            j = NBUF * jo + b
            jg = j + LG
            bg = (b + LG) % NBUF
            if b < LG:
                @pl.when((jo > 0) & (jg < CH_W))
                def _():
                    wait_scatter(bg)
            else:
                @pl.when(jg < CH_W)
                def _():
                    wait_scatter(bg)

            @pl.when(jg < CH_W)
            def _():
                issue_gather(jg, bg)

            wait_gather(b)
            issue_scatter(b)
        return carry

    lax.fori_loop(0, CH_W // NBUF, body, 0)
    for b in range(NBUF):
        wait_scatter(b)

    @pl.when(wid < EXTRA)
    def _():
        pltpu.sync_copy(ei_hbm.at[pl.ds((CH_W * NW + wid) * K, K)], sbufs[0])
        pltpu.async_copy(gs.at[sbufs[0]], rows[0], gsems[0]).wait()
        pltpu.sync_copy(ei_hbm.at[pl.ds(E + (CH_W * NW + wid) * K, K)], dbufs[0])
        pltpu.sync_copy(rows[0], acc_sp.at[dbufs[0]], add=True)

    plsc.subcore_barrier()
    pltpu.sync_copy(acc_sp.at[pl.ds(s * ROWS_T, ROWS_T), :],
                    accp.at[pl.ds(c * NPAD + s * ROWS_T, ROWS_T), :])


def _sc_pool(h4p, bip, zp, z24, poolp, cntw,
             b0, rows, ones_v, cv, cexp, pool_sp, cnt_sp):
    c = lax.axis_index("c")
    s = lax.axis_index("s")
    wid = c * NS + s
    pltpu.sync_copy(zp, pool_sp.at[pl.ds(s * BT, BT), :])
    pltpu.sync_copy(z24, cnt_sp.at[pl.ds(s * BT, BT)])
    _fill_ones(ones_v)
    plsc.subcore_barrier()

    def do_chunk(ch):
        base = ch * K
        pltpu.sync_copy(bip.at[pl.ds(base, K)], b0)
        pltpu.sync_copy(h4p.at[pl.ds(base, K), :], rows)
        pltpu.sync_copy(rows, pool_sp.at[b0], add=True)
        pltpu.sync_copy(ones_v, cnt_sp.at[b0], add=True)

    do_chunk(wid)
    do_chunk(wid + NW)

    @pl.when(wid < (NPAD // K) - 2 * NW)
    def _():
        do_chunk(wid + 2 * NW)

    plsc.subcore_barrier()
    pltpu.sync_copy(pool_sp.at[pl.ds(s * GT, GT), :],
                    poolp.at[pl.ds(c * NG + s * GT, GT), :])
    pltpu.sync_copy(cnt_sp.at[pl.ds(s * GT, GT)], cv)
    cvec = cv[pl.ds(0, LANES)]
    for i in range(GT):
        bv = jnp.full((LANES,), cvec[i], jnp.float32)
        for k in range(H // LANES):
            cexp[i, pl.ds(k * LANES, LANES)] = bv
    pltpu.sync_copy(cexp, cntw.at[pl.ds(c * NG + s * GT, GT), :])


def _tc_prep_mm0(dw_ref, xp_ref, wbd_ref, gs_ref, dvw_ref):
    dvw = lax.rsqrt(dw_ref[0:NP2, :] + dw_ref[NP2:2 * NP2, :] + 1.0)
    dvw_ref[...] = dvw
    gs_ref[pl.ds(0, XP), :] = dvw[:XP] * jnp.dot(
        xp_ref[...], wbd_ref[...], preferred_element_type=jnp.float32)
    gs_ref[pl.ds(XP, NP2 - XP), :] = jnp.zeros((NP2 - XP, HP), jnp.float32)


def _tc_layer(accp_ref, gsp_ref, dvw_ref, b_ref, wbd_ref, out_ref):
    a = accp_ref[0:NP2, :] + accp_ref[NP2:2 * NP2, :] - gsp_ref[...]
    h = jnp.maximum(dvw_ref[...] * a + b_ref[...], 0.0)
    out_ref[...] = dvw_ref[...] * jnp.dot(
        h, wbd_ref[...], preferred_element_type=jnp.float32)


def _tc_final(accp_ref, gsp_ref, dvw_ref, b_ref, out_ref):
    a = accp_ref[0:NP2, :] + accp_ref[NP2:2 * NP2, :] - gsp_ref[...]
    h = jnp.maximum(dvw_ref[...] * a + b_ref[...], 0.0)
    out_ref[...] = h
    out_ref[pl.ds(XP, NP2 - XP), :] = jnp.zeros((NP2 - XP, HP), jnp.float32)


def _tc_head(pp_ref, cw_ref, wbd_ref, b_ref, out_ref, hid_ref):
    sums = pp_ref[0:NGP, :] + pp_ref[NGP:2 * NGP, :]
    cnt = cw_ref[0:NGP, :] + cw_ref[NGP:2 * NGP, :]
    hid = sums / jnp.maximum(cnt, 1.0)
    hid_ref[...] = hid
    out_ref[...] = jnp.dot(
        hid, wbd_ref[...], preferred_element_type=jnp.float32) + b_ref[...]


def kernel(x, edge_index, batch_index, W0, b0, W1, b1, W2, b2, W3, b3,
           W_out, b_out):
    f32 = jnp.float32
    ei_flat = edge_index.reshape(2 * E)
    bip = jnp.concatenate(
        [batch_index, jnp.full((NPAD - N,), NG, jnp.int32)])
    z640 = jnp.zeros((DEG_T,), f32)
    zp = jnp.zeros((BT, H), f32)
    z24 = jnp.zeros((BT,), f32)

    def bd(W):
        fi, fo = W.shape
        z = jnp.zeros((fi, fo), f32)
        return jnp.concatenate(
            [jnp.concatenate([W, z], axis=1),
             jnp.concatenate([z, W], axis=1)], axis=0)

    deg_call = pl.kernel(
        _sc_deg,
        out_type=jax.ShapeDtypeStruct((NC * NPAD, H), f32),
        mesh=_mesh(),
        compiler_params=_SC_PARAMS,
        scratch_types=[
            pltpu.VMEM((EDGES_W,), jnp.int32),
            pltpu.VMEM((K,), jnp.int32),
            pltpu.VMEM((K,), f32),
            pltpu.VMEM((DEG_T,), f32),
            pltpu.VMEM((DEG_T, H), f32),
            pltpu.VMEM_SHARED((NPAD,), f32),
        ],
    )
    degw = deg_call(ei_flat, z640)

    gs, dvw = pl.pallas_call(
        _tc_prep_mm0,
        out_shape=(
            jax.ShapeDtypeStruct((NP2, HP), f32),
            jax.ShapeDtypeStruct((NP2, HP), f32),
        ),
    )(degw.reshape(2 * NP2, HP), x.reshape(XP, 2 * DIN), bd(W0))

    edge_call = pl.kernel(
        _sc_edge,
        out_type=jax.ShapeDtypeStruct((NC * NPAD, H), f32),
        mesh=_mesh(),
        compiler_params=_SC_PARAMS,
        scratch_types=[
            pltpu.VMEM((EDGES_W,), jnp.int32),
            pltpu.VMEM((EDGES_W,), jnp.int32),
            [pltpu.VMEM((K, H), f32) for _ in range(NBUF)],
            [pltpu.VMEM((K,), jnp.int32) for _ in range(NBUF)],
            [pltpu.VMEM((K,), jnp.int32) for _ in range(NBUF)],
            pltpu.VMEM_SHARED((NPAD, H), f32),
            [pltpu.SemaphoreType.DMA for _ in range(NBUF)],
            [pltpu.SemaphoreType.DMA for _ in range(NBUF)],
        ],
    )

    layer_call = pl.pallas_call(
        _tc_layer,
        out_shape=jax.ShapeDtypeStruct((NP2, HP), f32),
    )

    for W_l, b_l in ((W1, b0), (W2, b1), (W3, b2)):
        accp = edge_call(gs.reshape(NPAD, H), ei_flat)
        gs = layer_call(accp.reshape(2 * NP2, HP), gs,
                        dvw, jnp.tile(b_l, 2).reshape(1, HP), bd(W_l))

    accp = edge_call(gs.reshape(NPAD, H), ei_flat)
    h4p = pl.pallas_call(
        _tc_final,
        out_shape=jax.ShapeDtypeStruct((NP2, HP), f32),
    )(accp.reshape(2 * NP2, HP), gs, dvw, jnp.tile(b3, 2).reshape(1, HP))

    pool_call = pl.kernel(
        _sc_pool,
        out_type=(
            jax.ShapeDtypeStruct((NC * NG, H), f32),
            jax.ShapeDtypeStruct((NC * NG, H), f32),
        ),
        mesh=_mesh(),
        compiler_params=_SC_PARAMS,
        scratch_types=[
            pltpu.VMEM((K,), jnp.int32),
            pltpu.VMEM((K, H), f32),
            pltpu.VMEM((K,), f32),
            pltpu.VMEM((GT,), f32),
            pltpu.VMEM((GT, H), f32),
            pltpu.VMEM_SHARED((NB, H), f32),
            pltpu.VMEM_SHARED((NB,), f32),
        ],
    )
    poolp, cntw = pool_call(h4p.reshape(NPAD, H), bip, zp, z24)

    out_p, hid_p = pl.pallas_call(
        _tc_head,
        out_shape=(
            jax.ShapeDtypeStruct((NGP, 2), f32),
            jax.ShapeDtypeStruct((NGP, HP), f32),
        ),
    )(poolp.reshape(NG, HP), cntw.reshape(NG, HP), bd(W_out),
      jnp.tile(b_out, 2).reshape(1, 2))
    return (out_p.reshape(NG, 1), hid_p.reshape(NG, H))


# speedup vs baseline: 1.1056x; 1.0307x over previous
import jax
import jax.numpy as jnp
from jax import lax
from jax.experimental import pallas as pl
from jax.experimental.pallas import tpu as pltpu
from jax.experimental.pallas import tpu_sc as plsc

N = 10000
E = 320000
DIN = 128
H = 64
NG = 256

NC = 2
NS = 16
NW = NC * NS
LANES = 16

K = 128
CH_TOTAL = E // K
CH_W = CH_TOTAL // NW
EXTRA = CH_TOTAL - CH_W * NW
EDGES_W = CH_W * K

NPAD = 10240
DEG_T = NPAD // NS
ROWS_T = NPAD // NS
NB = 384
BT = NB // NS
GT = NG // NS

NP2 = NPAD // 2
XP = N // 2
HP = 2 * H
NGP = NG // 2


def _mesh():
    return plsc.VectorSubcoreMesh(core_axis_name="c", subcore_axis_name="s")


_SC_PARAMS = pltpu.CompilerParams(use_tc_tiling_on_sc=False)


def _fill_ones(ones_v):
    for k in range(K // LANES):
        ones_v[pl.ds(k * LANES, LANES)] = jnp.ones((LANES,), jnp.float32)


def _stage_chunk(src_ref, off, dst_ref):
    for k in range(K // LANES):
        dst_ref[pl.ds(k * LANES, LANES)] = src_ref[pl.ds(off + k * LANES, LANES)]


def _sc_deg(ei_hbm, z640, degw, d_all, d0, d1, ones_v, dv_v, dexp, deg_sp,
            dsem0, dsem1):
    c = lax.axis_index("c")
    s = lax.axis_index("s")
    wid = c * NS + s
    pltpu.sync_copy(ei_hbm.at[pl.ds(E + wid * EDGES_W, EDGES_W)], d_all)
    _fill_ones(ones_v)
    pltpu.sync_copy(z640, deg_sp.at[pl.ds(s * DEG_T, DEG_T)])
    plsc.subcore_barrier()

    dbufs = ((d0, dsem0), (d1, dsem1))

    def body(jo, carry):
        for b in (0, 1):
            d_b, sem_b = dbufs[b]
            j = 2 * jo + b

            @pl.when(jo > 0)
            def _():
                pltpu.make_async_copy(ones_v, deg_sp.at[d_b], sem_b).wait()

            _stage_chunk(d_all, j * K, d_b)
            pltpu.async_copy(ones_v, deg_sp.at[d_b], sem_b, add=True)
        return carry

    lax.fori_loop(0, CH_W // 2, body, 0)
    for d_b, sem_b in dbufs:
        pltpu.make_async_copy(ones_v, deg_sp.at[d_b], sem_b).wait()

    @pl.when(wid < EXTRA)
    def _():
        pltpu.sync_copy(ei_hbm.at[pl.ds(E + (CH_W * NW + wid) * K, K)], d0)
        pltpu.sync_copy(ones_v, deg_sp.at[d0], add=True)

    plsc.subcore_barrier()
    pltpu.sync_copy(deg_sp.at[pl.ds(s * DEG_T, DEG_T)], dv_v)

    def ebody(g, carry):
        vec = dv_v[pl.ds(g * LANES, LANES)]
        for l in range(LANES):
            bv = jnp.full((LANES,), vec[l], jnp.float32)
            for k in range(H // LANES):
                dexp[g * LANES + l, pl.ds(k * LANES, LANES)] = bv
        return carry

    lax.fori_loop(0, DEG_T // LANES, ebody, 0)
    pltpu.sync_copy(dexp, degw.at[pl.ds(c * NPAD + s * DEG_T, DEG_T), :])


NBUF = 6
LG = 3


def _sc_edge(gs, ei_hbm, accp,
             s_all, d_all, rows, sbufs, dbufs, acc_sp, gsems, ssems):
    c = lax.axis_index("c")
    s = lax.axis_index("s")
    wid = c * NS + s
    base_e = wid * EDGES_W
    pltpu.sync_copy(ei_hbm.at[pl.ds(base_e, EDGES_W)], s_all)
    pltpu.sync_copy(ei_hbm.at[pl.ds(E + base_e, EDGES_W)], d_all)

    def issue_gather(j, b):
        _stage_chunk(s_all, j * K, sbufs[b])
        _stage_chunk(d_all, j * K, dbufs[b])
        pltpu.async_copy(gs.at[sbufs[b]], rows[b], gsems[b])

    def wait_gather(b):
        pltpu.make_async_copy(gs.at[pl.ds(0, K), :], rows[b], gsems[b]).wait()

    def issue_scatter(b):
        pltpu.async_copy(rows[b], acc_sp.at[dbufs[b]], ssems[b], add=True)

    def wait_scatter(b):
        pltpu.make_async_copy(rows[b], acc_sp.at[dbufs[b]], ssems[b]).wait()

    for t in range(LG):
        issue_gather(t, t)

    pltpu.sync_copy(gs.at[pl.ds(s * ROWS_T, ROWS_T), :],
                    acc_sp.at[pl.ds(s * ROWS_T, ROWS_T), :])
    plsc.subcore_barrier()

    def body(jo, carry):
        for b in range(NBUF):
            j = NBUF * jo + b
            jg = j + LG
            bg = (b + LG) % NBUF
            if b < LG:
                @pl.when((jo > 0) & (jg < CH_W))
                def _():
                    wait_scatter(bg)
            else:
                @pl.when(jg < CH_W)
                def _():
                    wait_scatter(bg)

            @pl.when(jg < CH_W)
            def _():
                issue_gather(jg, bg)

            wait_gather(b)
            issue_scatter(b)
        return carry

    lax.fori_loop(0, CH_W // NBUF, body, 0)
    for b in range(NBUF):
        wait_scatter(b)

    @pl.when(wid < EXTRA)
    def _():
        pltpu.sync_copy(ei_hbm.at[pl.ds((CH_W * NW + wid) * K, K)], sbufs[0])
        pltpu.async_copy(gs.at[sbufs[0]], rows[0], gsems[0]).wait()
        pltpu.sync_copy(ei_hbm.at[pl.ds(E + (CH_W * NW + wid) * K, K)], dbufs[0])
        pltpu.sync_copy(rows[0], acc_sp.at[dbufs[0]], add=True)

    plsc.subcore_barrier()
    pltpu.sync_copy(acc_sp.at[pl.ds(s * ROWS_T, ROWS_T), :],
                    accp.at[pl.ds(c * NPAD + s * ROWS_T, ROWS_T), :])


def _sc_pool(h4p, bip, zp, z24, poolp, cntw,
             b0, rows, ones_v, cv, cexp, pool_sp, cnt_sp):
    c = lax.axis_index("c")
    s = lax.axis_index("s")
    wid = c * NS + s
    pltpu.sync_copy(zp, pool_sp.at[pl.ds(s * BT, BT), :])
    pltpu.sync_copy(z24, cnt_sp.at[pl.ds(s * BT, BT)])
    _fill_ones(ones_v)
    plsc.subcore_barrier()

    def do_chunk(ch):
        base = ch * K
        pltpu.sync_copy(bip.at[pl.ds(base, K)], b0)
        pltpu.sync_copy(h4p.at[pl.ds(base, K), :], rows)
        pltpu.sync_copy(rows, pool_sp.at[b0], add=True)
        pltpu.sync_copy(ones_v, cnt_sp.at[b0], add=True)

    do_chunk(wid)
    do_chunk(wid + NW)

    @pl.when(wid < (NPAD // K) - 2 * NW)
    def _():
        do_chunk(wid + 2 * NW)

    plsc.subcore_barrier()
    pltpu.sync_copy(pool_sp.at[pl.ds(s * GT, GT), :],
                    poolp.at[pl.ds(c * NG + s * GT, GT), :])
    pltpu.sync_copy(cnt_sp.at[pl.ds(s * GT, GT)], cv)
    cvec = cv[pl.ds(0, LANES)]
    for i in range(GT):
        bv = jnp.full((LANES,), cvec[i], jnp.float32)
        for k in range(H // LANES):
            cexp[i, pl.ds(k * LANES, LANES)] = bv
    pltpu.sync_copy(cexp, cntw.at[pl.ds(c * NG + s * GT, GT), :])


def _tc_prep_mm0(dw_ref, xp_ref, wbd_ref, gs_ref, dvw_ref):
    dvw = lax.rsqrt(dw_ref[0:NP2, :] + dw_ref[NP2:2 * NP2, :] + 1.0)
    dvw_ref[...] = dvw
    gs_ref[pl.ds(0, XP), :] = dvw[:XP] * jnp.dot(
        xp_ref[...], wbd_ref[...], preferred_element_type=jnp.float32)
    gs_ref[pl.ds(XP, NP2 - XP), :] = jnp.zeros((NP2 - XP, HP), jnp.float32)


def _tc_layer(accp_ref, gsp_ref, dvw_ref, b_ref, wbd_ref, out_ref):
    a = accp_ref[0:NP2, :] + accp_ref[NP2:2 * NP2, :] - gsp_ref[...]
    h = jnp.maximum(dvw_ref[...] * a + b_ref[...], 0.0)
    out_ref[...] = dvw_ref[...] * jnp.dot(
        h, wbd_ref[...], preferred_element_type=jnp.float32)


def _tc_final(accp_ref, gsp_ref, dvw_ref, b_ref, out_ref):
    a = accp_ref[0:NP2, :] + accp_ref[NP2:2 * NP2, :] - gsp_ref[...]
    h = jnp.maximum(dvw_ref[...] * a + b_ref[...], 0.0)
    out_ref[...] = h
    out_ref[pl.ds(XP, NP2 - XP), :] = jnp.zeros((NP2 - XP, HP), jnp.float32)


def _tc_head(pp_ref, cw_ref, wbd_ref, b_ref, out_ref, hid_ref):
    sums = pp_ref[0:NGP, :] + pp_ref[NGP:2 * NGP, :]
    cnt = cw_ref[0:NGP, :] + cw_ref[NGP:2 * NGP, :]
    hid = sums / jnp.maximum(cnt, 1.0)
    hid_ref[...] = hid
    out_ref[...] = jnp.dot(
        hid, wbd_ref[...], preferred_element_type=jnp.float32) + b_ref[...]


def kernel(x, edge_index, batch_index, W0, b0, W1, b1, W2, b2, W3, b3,
           W_out, b_out):
    f32 = jnp.float32
    ei_flat = edge_index.reshape(2 * E)
    bip = jnp.concatenate(
        [batch_index, jnp.full((NPAD - N,), NG, jnp.int32)])
    z640 = jnp.zeros((DEG_T,), f32)
    zp = jnp.zeros((BT, H), f32)
    z24 = jnp.zeros((BT,), f32)

    def bd(W):
        fi, fo = W.shape
        z = jnp.zeros((fi, fo), f32)
        return jnp.concatenate(
            [jnp.concatenate([W, z], axis=1),
             jnp.concatenate([z, W], axis=1)], axis=0)

    deg_call = pl.kernel(
        _sc_deg,
        out_type=jax.ShapeDtypeStruct((NC * NPAD, H), f32),
        mesh=_mesh(),
        compiler_params=_SC_PARAMS,
        scratch_types=[
            pltpu.VMEM((EDGES_W,), jnp.int32),
            pltpu.VMEM((K,), jnp.int32),
            pltpu.VMEM((K,), jnp.int32),
            pltpu.VMEM((K,), f32),
            pltpu.VMEM((DEG_T,), f32),
            pltpu.VMEM((DEG_T, H), f32),
            pltpu.VMEM_SHARED((NPAD,), f32),
            pltpu.SemaphoreType.DMA,
            pltpu.SemaphoreType.DMA,
        ],
    )
    degw = deg_call(ei_flat, z640)

    gs, dvw = pl.pallas_call(
        _tc_prep_mm0,
        out_shape=(
            jax.ShapeDtypeStruct((NP2, HP), f32),
            jax.ShapeDtypeStruct((NP2, HP), f32),
        ),
    )(degw.reshape(2 * NP2, HP), x.reshape(XP, 2 * DIN), bd(W0))

    edge_call = pl.kernel(
        _sc_edge,
        out_type=jax.ShapeDtypeStruct((NC * NPAD, H), f32),
        mesh=_mesh(),
        compiler_params=_SC_PARAMS,
        scratch_types=[
            pltpu.VMEM((EDGES_W,), jnp.int32),
            pltpu.VMEM((EDGES_W,), jnp.int32),
            [pltpu.VMEM((K, H), f32) for _ in range(NBUF)],
            [pltpu.VMEM((K,), jnp.int32) for _ in range(NBUF)],
            [pltpu.VMEM((K,), jnp.int32) for _ in range(NBUF)],
            pltpu.VMEM_SHARED((NPAD, H), f32),
            [pltpu.SemaphoreType.DMA for _ in range(NBUF)],
            [pltpu.SemaphoreType.DMA for _ in range(NBUF)],
        ],
    )

    layer_call = pl.pallas_call(
        _tc_layer,
        out_shape=jax.ShapeDtypeStruct((NP2, HP), f32),
    )

    for W_l, b_l in ((W1, b0), (W2, b1), (W3, b2)):
        accp = edge_call(gs.reshape(NPAD, H), ei_flat)
        gs = layer_call(accp.reshape(2 * NP2, HP), gs,
                        dvw, jnp.tile(b_l, 2).reshape(1, HP), bd(W_l))

    accp = edge_call(gs.reshape(NPAD, H), ei_flat)
    h4p = pl.pallas_call(
        _tc_final,
        out_shape=jax.ShapeDtypeStruct((NP2, HP), f32),
    )(accp.reshape(2 * NP2, HP), gs, dvw, jnp.tile(b3, 2).reshape(1, HP))

    pool_call = pl.kernel(
        _sc_pool,
        out_type=(
            jax.ShapeDtypeStruct((NC * NG, H), f32),
            jax.ShapeDtypeStruct((NC * NG, H), f32),
        ),
        mesh=_mesh(),
        compiler_params=_SC_PARAMS,
        scratch_types=[
            pltpu.VMEM((K,), jnp.int32),
            pltpu.VMEM((K, H), f32),
            pltpu.VMEM((K,), f32),
            pltpu.VMEM((GT,), f32),
            pltpu.VMEM((GT, H), f32),
            pltpu.VMEM_SHARED((NB, H), f32),
            pltpu.VMEM_SHARED((NB,), f32),
        ],
    )
    poolp, cntw = pool_call(h4p.reshape(NPAD, H), bip, zp, z24)

    out_p, hid_p = pl.pallas_call(
        _tc_head,
        out_shape=(
            jax.ShapeDtypeStruct((NGP, 2), f32),
            jax.ShapeDtypeStruct((NGP, HP), f32),
        ),
    )(poolp.reshape(NG, HP), cntw.reshape(NG, HP), bd(W_out),
      jnp.tile(b_out, 2).reshape(1, 2))
    return (out_p.reshape(NG, 1), hid_p.reshape(NG, H))


# speedup vs baseline: 1.1156x; 1.0091x over previous
import jax
import jax.numpy as jnp
from jax import lax
from jax.experimental import pallas as pl
from jax.experimental.pallas import tpu as pltpu
from jax.experimental.pallas import tpu_sc as plsc

N = 10000
E = 320000
DIN = 128
H = 64
NG = 256

NC = 2
NS = 16
NW = NC * NS
LANES = 16

K = 128
CH_TOTAL = E // K
CH_W = CH_TOTAL // NW
EXTRA = CH_TOTAL - CH_W * NW
EDGES_W = CH_W * K

NPAD = 10240
DEG_T = NPAD // NS
ROWS_T = NPAD // NS
NB = 384
BT = NB // NS
GT = NG // NS

NP2 = NPAD // 2
XP = N // 2
HP = 2 * H
NGP = NG // 2


def _mesh():
    return plsc.VectorSubcoreMesh(core_axis_name="c", subcore_axis_name="s")


_SC_PARAMS = pltpu.CompilerParams(use_tc_tiling_on_sc=False)


def _fill_ones(ones_v):
    for k in range(K // LANES):
        ones_v[pl.ds(k * LANES, LANES)] = jnp.ones((LANES,), jnp.float32)


def _stage_chunk(src_ref, off, dst_ref):
    for k in range(K // LANES):
        dst_ref[pl.ds(k * LANES, LANES)] = src_ref[pl.ds(off + k * LANES, LANES)]


def _sc_deg(ei_hbm, z640, degw, d_all, d0, d1, ones_v, dv_v, dexp, deg_sp,
            dsem0, dsem1):
    c = lax.axis_index("c")
    s = lax.axis_index("s")
    wid = c * NS + s
    pltpu.sync_copy(ei_hbm.at[pl.ds(E + wid * EDGES_W, EDGES_W)], d_all)
    _fill_ones(ones_v)
    pltpu.sync_copy(z640, deg_sp.at[pl.ds(s * DEG_T, DEG_T)])
    plsc.subcore_barrier()

    dbufs = ((d0, dsem0), (d1, dsem1))

    def body(jo, carry):
        for b in (0, 1):
            d_b, sem_b = dbufs[b]
            j = 2 * jo + b

            @pl.when(jo > 0)
            def _():
                pltpu.make_async_copy(ones_v, deg_sp.at[d_b], sem_b).wait()

            _stage_chunk(d_all, j * K, d_b)
            pltpu.async_copy(ones_v, deg_sp.at[d_b], sem_b, add=True)
        return carry

    lax.fori_loop(0, CH_W // 2, body, 0)
    for d_b, sem_b in dbufs:
        pltpu.make_async_copy(ones_v, deg_sp.at[d_b], sem_b).wait()

    @pl.when(wid < EXTRA)
    def _():
        pltpu.sync_copy(ei_hbm.at[pl.ds(E + (CH_W * NW + wid) * K, K)], d0)
        pltpu.sync_copy(ones_v, deg_sp.at[d0], add=True)

    plsc.subcore_barrier()
    pltpu.sync_copy(deg_sp.at[pl.ds(s * DEG_T, DEG_T)], dv_v)

    def ebody(g, carry):
        vec = dv_v[pl.ds(g * LANES, LANES)]
        for l in range(LANES):
            bv = jnp.full((LANES,), vec[l], jnp.float32)
            for k in range(H // LANES):
                dexp[g * LANES + l, pl.ds(k * LANES, LANES)] = bv
        return carry

    lax.fori_loop(0, DEG_T // LANES, ebody, 0)
    pltpu.sync_copy(dexp, degw.at[pl.ds(c * NPAD + s * DEG_T, DEG_T), :])


NBUF = 6
LG = 3


def _sc_edge(gs, ei_hbm, accp,
             s_all, d_all, rows, sbufs, dbufs, acc_sp, gsems, ssems):
    c = lax.axis_index("c")
    s = lax.axis_index("s")
    wid = c * NS + s
    base_e = wid * EDGES_W
    pltpu.sync_copy(ei_hbm.at[pl.ds(base_e, EDGES_W)], s_all)
    pltpu.sync_copy(ei_hbm.at[pl.ds(E + base_e, EDGES_W)], d_all)

    def issue_gather(j, b):
        _stage_chunk(s_all, j * K, sbufs[b])
        _stage_chunk(d_all, j * K, dbufs[b])
        pltpu.async_copy(gs.at[sbufs[b]], rows[b], gsems[b])

    def wait_gather(b):
        pltpu.make_async_copy(gs.at[pl.ds(0, K), :], rows[b], gsems[b]).wait()

    def issue_scatter(b):
        pltpu.async_copy(rows[b], acc_sp.at[dbufs[b]], ssems[b], add=True)

    def wait_scatter(b):
        pltpu.make_async_copy(rows[b], acc_sp.at[dbufs[b]], ssems[b]).wait()

    for t in range(LG):
        issue_gather(t, t)

    pltpu.sync_copy(gs.at[pl.ds(s * ROWS_T, ROWS_T), :],
                    acc_sp.at[pl.ds(s * ROWS_T, ROWS_T), :])
    plsc.subcore_barrier()

    def body(jo, carry):
        for b in range(NBUF):
            j = NBUF * jo + b
            jg = j + LG
            bg = (b + LG) % NBUF
            if b < LG:
                @pl.when((jo > 0) & (jg < CH_W))
                def _():
                    wait_scatter(bg)
            else:
                @pl.when(jg < CH_W)
                def _():
                    wait_scatter(bg)

            @pl.when(jg < CH_W)
            def _():
                issue_gather(jg, bg)

            wait_gather(b)
            issue_scatter(b)
        return carry

    lax.fori_loop(0, CH_W // NBUF, body, 0)
    for b in range(NBUF):
        wait_scatter(b)

    @pl.when(wid < EXTRA)
    def _():
        pltpu.sync_copy(ei_hbm.at[pl.ds((CH_W * NW + wid) * K, K)], sbufs[0])
        pltpu.async_copy(gs.at[sbufs[0]], rows[0], gsems[0]).wait()
        pltpu.sync_copy(ei_hbm.at[pl.ds(E + (CH_W * NW + wid) * K, K)], dbufs[0])
        pltpu.sync_copy(rows[0], acc_sp.at[dbufs[0]], add=True)

    plsc.subcore_barrier()
    pltpu.sync_copy(acc_sp.at[pl.ds(s * ROWS_T, ROWS_T), :],
                    accp.at[pl.ds(c * NPAD + s * ROWS_T, ROWS_T), :])


def _sc_pool(h4p, bip, zp, z24, poolp, cntw,
             b0, rows, ones_v, cv, cexp, pool_sp, cnt_sp):
    c = lax.axis_index("c")
    s = lax.axis_index("s")
    wid = c * NS + s
    pltpu.sync_copy(zp, pool_sp.at[pl.ds(s * BT, BT), :])
    pltpu.sync_copy(z24, cnt_sp.at[pl.ds(s * BT, BT)])
    _fill_ones(ones_v)
    plsc.subcore_barrier()

    def do_chunk(ch):
        base = ch * K
        pltpu.sync_copy(bip.at[pl.ds(base, K)], b0)
        pltpu.sync_copy(h4p.at[pl.ds(base, K), :], rows)
        pltpu.sync_copy(rows, pool_sp.at[b0], add=True)
        pltpu.sync_copy(ones_v, cnt_sp.at[b0], add=True)

    do_chunk(wid)
    do_chunk(wid + NW)

    @pl.when(wid < (NPAD // K) - 2 * NW)
    def _():
        do_chunk(wid + 2 * NW)

    plsc.subcore_barrier()
    pltpu.sync_copy(pool_sp.at[pl.ds(s * GT, GT), :],
                    poolp.at[pl.ds(c * NG + s * GT, GT), :])
    pltpu.sync_copy(cnt_sp.at[pl.ds(s * GT, GT)], cv)
    cvec = cv[pl.ds(0, LANES)]
    for i in range(GT):
        bv = jnp.full((LANES,), cvec[i], jnp.float32)
        for k in range(H // LANES):
            cexp[i, pl.ds(k * LANES, LANES)] = bv
    pltpu.sync_copy(cexp, cntw.at[pl.ds(c * NG + s * GT, GT), :])


def _tc_prep_mm0(dw_ref, xp_ref, wbd_ref, gs_ref, dvw_ref):
    dvw = lax.rsqrt(dw_ref[0:NP2, :] + dw_ref[NP2:2 * NP2, :] + 1.0)
    dvw_ref[...] = dvw
    gs_ref[pl.ds(0, XP), :] = dvw[:XP] * jnp.dot(
        xp_ref[...], wbd_ref[...], preferred_element_type=jnp.float32)
    gs_ref[pl.ds(XP, NP2 - XP), :] = jnp.zeros((NP2 - XP, HP), jnp.float32)


RB = NP2 // 2


def _tc_layer(acc0_ref, acc1_ref, gsp_ref, dvw_ref, b_ref, wbd_ref, out_ref):
    a = acc0_ref[...] + acc1_ref[...] - gsp_ref[...]
    h = jnp.maximum(dvw_ref[...] * a + b_ref[...], 0.0)
    out_ref[...] = dvw_ref[...] * jnp.dot(
        h, wbd_ref[...], preferred_element_type=jnp.float32)


_LAYER_GRID = dict(
    grid=(2,),
    in_specs=[
        pl.BlockSpec((RB, HP), lambda i: (i, 0)),
        pl.BlockSpec((RB, HP), lambda i: (i + 2, 0)),
        pl.BlockSpec((RB, HP), lambda i: (i, 0)),
        pl.BlockSpec((RB, HP), lambda i: (i, 0)),
        pl.BlockSpec((1, HP), lambda i: (0, 0)),
        pl.BlockSpec((HP, HP), lambda i: (0, 0)),
    ],
    out_specs=pl.BlockSpec((RB, HP), lambda i: (i, 0)),
)


def _tc_final(accp_ref, gsp_ref, dvw_ref, b_ref, out_ref):
    a = accp_ref[0:NP2, :] + accp_ref[NP2:2 * NP2, :] - gsp_ref[...]
    h = jnp.maximum(dvw_ref[...] * a + b_ref[...], 0.0)
    out_ref[...] = h
    out_ref[pl.ds(XP, NP2 - XP), :] = jnp.zeros((NP2 - XP, HP), jnp.float32)


def _tc_head(pp_ref, cw_ref, wbd_ref, b_ref, out_ref, hid_ref):
    sums = pp_ref[0:NGP, :] + pp_ref[NGP:2 * NGP, :]
    cnt = cw_ref[0:NGP, :] + cw_ref[NGP:2 * NGP, :]
    hid = sums / jnp.maximum(cnt, 1.0)
    hid_ref[...] = hid
    out_ref[...] = jnp.dot(
        hid, wbd_ref[...], preferred_element_type=jnp.float32) + b_ref[...]


def kernel(x, edge_index, batch_index, W0, b0, W1, b1, W2, b2, W3, b3,
           W_out, b_out):
    f32 = jnp.float32
    ei_flat = edge_index.reshape(2 * E)
    bip = jnp.concatenate(
        [batch_index, jnp.full((NPAD - N,), NG, jnp.int32)])
    z640 = jnp.zeros((DEG_T,), f32)
    zp = jnp.zeros((BT, H), f32)
    z24 = jnp.zeros((BT,), f32)

    def bd(W):
        fi, fo = W.shape
        z = jnp.zeros((fi, fo), f32)
        return jnp.concatenate(
            [jnp.concatenate([W, z], axis=1),
             jnp.concatenate([z, W], axis=1)], axis=0)

    deg_call = pl.kernel(
        _sc_deg,
        out_type=jax.ShapeDtypeStruct((NC * NPAD, H), f32),
        mesh=_mesh(),
        compiler_params=_SC_PARAMS,
        scratch_types=[
            pltpu.VMEM((EDGES_W,), jnp.int32),
            pltpu.VMEM((K,), jnp.int32),
            pltpu.VMEM((K,), jnp.int32),
            pltpu.VMEM((K,), f32),
            pltpu.VMEM((DEG_T,), f32),
            pltpu.VMEM((DEG_T, H), f32),
            pltpu.VMEM_SHARED((NPAD,), f32),
            pltpu.SemaphoreType.DMA,
            pltpu.SemaphoreType.DMA,
        ],
    )
    degw = deg_call(ei_flat, z640)

    gs, dvw = pl.pallas_call(
        _tc_prep_mm0,
        out_shape=(
            jax.ShapeDtypeStruct((NP2, HP), f32),
            jax.ShapeDtypeStruct((NP2, HP), f32),
        ),
    )(degw.reshape(2 * NP2, HP), x.reshape(XP, 2 * DIN), bd(W0))

    edge_call = pl.kernel(
        _sc_edge,
        out_type=jax.ShapeDtypeStruct((NC * NPAD, H), f32),
        mesh=_mesh(),
        compiler_params=_SC_PARAMS,
        scratch_types=[
            pltpu.VMEM((EDGES_W,), jnp.int32),
            pltpu.VMEM((EDGES_W,), jnp.int32),
            [pltpu.VMEM((K, H), f32) for _ in range(NBUF)],
            [pltpu.VMEM((K,), jnp.int32) for _ in range(NBUF)],
            [pltpu.VMEM((K,), jnp.int32) for _ in range(NBUF)],
            pltpu.VMEM_SHARED((NPAD, H), f32),
            [pltpu.SemaphoreType.DMA for _ in range(NBUF)],
            [pltpu.SemaphoreType.DMA for _ in range(NBUF)],
        ],
    )

    layer_call = pl.pallas_call(
        _tc_layer,
        out_shape=jax.ShapeDtypeStruct((NP2, HP), f32),
        **_LAYER_GRID,
    )

    for W_l, b_l in ((W1, b0), (W2, b1), (W3, b2)):
        accp = edge_call(gs.reshape(NPAD, H), ei_flat).reshape(2 * NP2, HP)
        gs = layer_call(accp, accp, gs,
                        dvw, jnp.tile(b_l, 2).reshape(1, HP), bd(W_l))

    accp = edge_call(gs.reshape(NPAD, H), ei_flat)
    h4p = pl.pallas_call(
        _tc_final,
        out_shape=jax.ShapeDtypeStruct((NP2, HP), f32),
    )(accp.reshape(2 * NP2, HP), gs, dvw, jnp.tile(b3, 2).reshape(1, HP))

    pool_call = pl.kernel(
        _sc_pool,
        out_type=(
            jax.ShapeDtypeStruct((NC * NG, H), f32),
            jax.ShapeDtypeStruct((NC * NG, H), f32),
        ),
        mesh=_mesh(),
        compiler_params=_SC_PARAMS,
        scratch_types=[
            pltpu.VMEM((K,), jnp.int32),
            pltpu.VMEM((K, H), f32),
            pltpu.VMEM((K,), f32),
            pltpu.VMEM((GT,), f32),
            pltpu.VMEM((GT, H), f32),
            pltpu.VMEM_SHARED((NB, H), f32),
            pltpu.VMEM_SHARED((NB,), f32),
        ],
    )
    poolp, cntw = pool_call(h4p.reshape(NPAD, H), bip, zp, z24)

    out_p, hid_p = pl.pallas_call(
        _tc_head,
        out_shape=(
            jax.ShapeDtypeStruct((NGP, 2), f32),
            jax.ShapeDtypeStruct((NGP, HP), f32),
        ),
    )(poolp.reshape(NG, HP), cntw.reshape(NG, HP), bd(W_out),
      jnp.tile(b_out, 2).reshape(1, 2))
    return (out_p.reshape(NG, 1), hid_p.reshape(NG, H))
